# baseline (speedup 1.0000x reference)
import jax
import jax.numpy as jnp
from jax.experimental import pallas as pl
from jax.experimental.pallas import tpu as pltpu

_S, _D, _SD = 2048, 1024, 64
_ND, _NS, _NC = 4, 4, 4
_NB, _BSZ = 4, 16
_K = _ND * _NS * _NC


def _group_mats(width, group):
    i = jax.lax.broadcasted_iota(jnp.int32, (width, width), 0)
    j = jax.lax.broadcasted_iota(jnp.int32, (width, width), 1)
    g = jnp.where((i // group) == (j // group), 1.0, 0.0).astype(jnp.bfloat16)
    gi = jax.lax.broadcasted_iota(jnp.int32, (width // group, width), 0)
    gj = jax.lax.broadcasted_iota(jnp.int32, (width // group, width), 1)
    b = jnp.where(gi == (gj // group), 1.0, 0.0).astype(jnp.bfloat16)
    return g, b


def _body(x_ref, wcol_ref, bcol_ref, wb1_ref, bb1_ref, bw1_ref,
          wb2_ref, bb2_ref, bw2_ref, wb3_ref, bb3_ref, bw3_ref,
          we_ref, be_ref, gamma_ref, beta_ref,
          out_ref, xb_scr, j3_scr, cols_scr):
    k = pl.program_id(0)

    @pl.when(k == 0)
    def _routing():
        xb = x_ref[...].astype(jnp.bfloat16)
        xb_scr[...] = x_ref[...].astype(jnp.float8_e4m3fn)
        wcol = wcol_ref[...].astype(jnp.bfloat16)
        color = jnp.tanh(
            jnp.dot(xb, wcol, preferred_element_type=jnp.float32)
            + bcol_ref[...])
        color_b = color.astype(jnp.bfloat16)

        def chroma(wb_ref, bb_ref, bw_ref, kl):
            bw = bw_ref[...]
            e = jnp.exp(bw - jnp.max(bw, axis=-1, keepdims=True))
            w = e / jnp.sum(e, axis=-1, keepdims=True)
            r = jnp.zeros((_S, kl), jnp.float32)
            for n in range(_NB):
                band = color_b[:, n * _BSZ:(n + 1) * _BSZ]
                wn = wb_ref[n].astype(jnp.bfloat16)
                logit = jnp.dot(band, wn, preferred_element_type=jnp.float32)
                logit = logit + bb_ref[n:n + 1, :]
                r = r + w[:, n:n + 1] * jax.nn.sigmoid(logit)
            return r

        r1 = chroma(wb1_ref, bb1_ref, bw1_ref, _ND)
        r2 = chroma(wb2_ref, bb2_ref, bw2_ref, _ND * _NS)
        r3 = chroma(wb3_ref, bb3_ref, bw3_ref, _K)

        e1 = jnp.exp(r1)
        p1 = e1 / jnp.sum(e1, axis=-1, keepdims=True)

        g16, b16 = _group_mats(_ND * _NS, _NS)
        e2 = jnp.exp(r2)
        s2 = jnp.dot(e2.astype(jnp.bfloat16), g16, preferred_element_type=jnp.float32)
        p2 = e2 / s2
        j2 = jnp.dot(p1.astype(jnp.bfloat16), b16, preferred_element_type=jnp.float32) * p2

        g64, b64 = _group_mats(_K, _NC)
        e3 = jnp.exp(r3)
        s3 = jnp.dot(e3.astype(jnp.bfloat16), g64, preferred_element_type=jnp.float32)
        p3 = e3 / s3
        j3 = jnp.dot(j2.astype(jnp.bfloat16), b64, preferred_element_type=jnp.float32) * p3
        j3_scr[...] = j3.astype(jnp.bfloat16)
        out_ref[...] = jnp.zeros((_S, _D), jnp.float32)

    def _col(idx):
        oh = (jax.lax.broadcasted_iota(jnp.int32, (_K, 1), 0) == idx
              ).astype(jnp.bfloat16)
        return jnp.dot(j3_scr[...], oh, preferred_element_type=jnp.float32)

    we = we_ref[0].astype(jnp.float8_e4m3fn)
    ph = jax.lax.rem(k, 8)

    @pl.when(ph == 0)
    def _cols():
        ri = jax.lax.broadcasted_iota(jnp.int32, (_K, 8), 0)
        ci = jax.lax.broadcasted_iota(jnp.int32, (_K, 8), 1)
        oh8 = (ri == k + ci).astype(jnp.bfloat16)
        cols_scr[...] = jnp.dot(j3_scr[...], oh8,
                                preferred_element_type=jnp.float32)

    mm = jnp.dot(xb_scr[...], we, preferred_element_type=jnp.float32)
    for i0 in range(8):
        @pl.when(ph == i0)
        def _combine(i0=i0):
            out_ref[...] = out_ref[...] + cols_scr[:, i0:i0 + 1] * mm

    @pl.when(k == _K - 1)
    def _finish():
        h = out_ref[...] + jnp.dot(
            j3_scr[...], be_ref[...].astype(jnp.bfloat16),
            preferred_element_type=jnp.float32)
        y = x_ref[...] + jax.nn.gelu(h)
        mu = jnp.mean(y, axis=-1, keepdims=True)
        var = jnp.mean((y - mu) ** 2, axis=-1, keepdims=True)
        out_ref[...] = ((y - mu) * jax.lax.rsqrt(var + 1e-5)
                        * gamma_ref[...] + beta_ref[...])


def kernel(x, W_color, b_color, Wb1, bb1, bw1, Wb2, bb2, bw2, Wb3, bb3, bw3,
           We, be, gamma, beta):
    x2 = x.reshape(_S, _D)
    out = pl.pallas_call(
        _body,
        grid=(_K,),
        in_specs=[
            pl.BlockSpec((_S, _D), lambda k: (0, 0)),
            pl.BlockSpec((_D, _SD), lambda k: (0, 0)),
            pl.BlockSpec((1, _SD), lambda k: (0, 0)),
            pl.BlockSpec((_NB, _BSZ, _ND), lambda k: (0, 0, 0)),
            pl.BlockSpec((_NB, _ND), lambda k: (0, 0)),
            pl.BlockSpec((1, _NB), lambda k: (0, 0)),
            pl.BlockSpec((_NB, _BSZ, _ND * _NS), lambda k: (0, 0, 0)),
            pl.BlockSpec((_NB, _ND * _NS), lambda k: (0, 0)),
            pl.BlockSpec((1, _NB), lambda k: (0, 0)),
            pl.BlockSpec((_NB, _BSZ, _K), lambda k: (0, 0, 0)),
            pl.BlockSpec((_NB, _K), lambda k: (0, 0)),
            pl.BlockSpec((1, _NB), lambda k: (0, 0)),
            pl.BlockSpec((1, _D, _D), lambda k: (k, 0, 0)),
            pl.BlockSpec((_K, _D), lambda k: (0, 0)),
            pl.BlockSpec((1, _D), lambda k: (0, 0)),
            pl.BlockSpec((1, _D), lambda k: (0, 0)),
        ],
        out_specs=pl.BlockSpec((_S, _D), lambda k: (0, 0)),
        out_shape=jax.ShapeDtypeStruct((_S, _D), jnp.float32),
        scratch_shapes=[
            pltpu.VMEM((_S, _D), jnp.float8_e4m3fn),
            pltpu.VMEM((_S, _K), jnp.bfloat16),
            pltpu.VMEM((_S, 8), jnp.float32),
        ],
        compiler_params=pltpu.CompilerParams(
            dimension_semantics=("arbitrary",),
            vmem_limit_bytes=63 * 1024 * 1024,
        ),
    )(x2, W_color, b_color.reshape(1, _SD), Wb1, bb1, bw1.reshape(1, _NB),
      Wb2, bb2, bw2.reshape(1, _NB), Wb3, bb3, bw3.reshape(1, _NB),
      We, be, gamma.reshape(1, _D), beta.reshape(1, _D))
    return out.reshape(1, _S, _D)


# speedup vs baseline: 1.1251x; 1.1251x over previous
import jax
import jax.numpy as jnp
from jax.experimental import pallas as pl
from jax.experimental.pallas import tpu as pltpu

_S, _D, _SD = 2048, 1024, 64
_ND, _NS, _NC = 4, 4, 4
_NB, _BSZ = 4, 16
_K = _ND * _NS * _NC


def _group_mats(width, group):
    i = jax.lax.broadcasted_iota(jnp.int32, (width, width), 0)
    j = jax.lax.broadcasted_iota(jnp.int32, (width, width), 1)
    g = jnp.where((i // group) == (j // group), 1.0, 0.0).astype(jnp.bfloat16)
    gi = jax.lax.broadcasted_iota(jnp.int32, (width // group, width), 0)
    gj = jax.lax.broadcasted_iota(jnp.int32, (width // group, width), 1)
    b = jnp.where(gi == (gj // group), 1.0, 0.0).astype(jnp.bfloat16)
    return g, b


def _body(x_ref, wcol_ref, bcol_ref, wb1_ref, bb1_ref, bw1_ref,
          wb2_ref, bb2_ref, bw2_ref, wb3_ref, bb3_ref, bw3_ref,
          we_ref, bet_ref, gamma_ref, beta_ref, xt_ref,
          out_ref, xt8_scr, j3t_scr, j3rep_scr):
    k = pl.program_id(0)

    @pl.when(k == 0)
    def _routing():
        xb = x_ref[...].astype(jnp.bfloat16)
        xt8_scr[...] = xt_ref[...].astype(jnp.float8_e4m3fn)
        wcol = wcol_ref[...].astype(jnp.bfloat16)
        color = jnp.tanh(
            jnp.dot(xb, wcol, preferred_element_type=jnp.float32)
            + bcol_ref[...])
        color_b = color.astype(jnp.bfloat16)

        def chroma(wb_ref, bb_ref, bw_ref, kl):
            bw = bw_ref[...]
            e = jnp.exp(bw - jnp.max(bw, axis=-1, keepdims=True))
            w = e / jnp.sum(e, axis=-1, keepdims=True)
            r = jnp.zeros((_S, kl), jnp.float32)
            for n in range(_NB):
                band = color_b[:, n * _BSZ:(n + 1) * _BSZ]
                wn = wb_ref[n].astype(jnp.bfloat16)
                logit = jnp.dot(band, wn, preferred_element_type=jnp.float32)
                logit = logit + bb_ref[n:n + 1, :]
                r = r + w[:, n:n + 1] * jax.nn.sigmoid(logit)
            return r

        r1 = chroma(wb1_ref, bb1_ref, bw1_ref, _ND)
        r2 = chroma(wb2_ref, bb2_ref, bw2_ref, _ND * _NS)
        r3 = chroma(wb3_ref, bb3_ref, bw3_ref, _K)

        e1 = jnp.exp(r1)
        p1 = e1 / jnp.sum(e1, axis=-1, keepdims=True)

        g16, b16 = _group_mats(_ND * _NS, _NS)
        e2 = jnp.exp(r2)
        s2 = jnp.dot(e2.astype(jnp.bfloat16), g16, preferred_element_type=jnp.float32)
        p2 = e2 / s2
        j2 = jnp.dot(p1.astype(jnp.bfloat16), b16, preferred_element_type=jnp.float32) * p2

        g64, b64 = _group_mats(_K, _NC)
        e3 = jnp.exp(r3)
        s3 = jnp.dot(e3.astype(jnp.bfloat16), g64, preferred_element_type=jnp.float32)
        p3 = e3 / s3
        j3 = jnp.dot(j2.astype(jnp.bfloat16), b64, preferred_element_type=jnp.float32) * p3
        j3t = jnp.transpose(j3)
        j3t_scr[...] = j3t.astype(jnp.bfloat16)
        for r in range(8):
            j3rep_scr[:, r, :] = j3t
        out_ref[...] = jnp.zeros((_D, _S), jnp.float32)

    we8 = we_ref[0].astype(jnp.float8_e4m3fn)
    mmt = jax.lax.dot_general(
        we8, xt8_scr[...],
        dimension_numbers=(((0,), (0,)), ((), ())),
        preferred_element_type=jnp.float32)
    row = j3rep_scr[pl.ds(k, 1)][0, 0:1, :]
    out_ref[...] = out_ref[...] + row * mmt

    @pl.when(k == _K - 1)
    def _finish():
        ht = out_ref[...] + jnp.dot(
            bet_ref[...].astype(jnp.bfloat16), j3t_scr[...],
            preferred_element_type=jnp.float32)
        yt = xt_ref[...] + jax.nn.gelu(ht)
        mu = jnp.mean(yt, axis=0, keepdims=True)
        var = jnp.mean((yt - mu) ** 2, axis=0, keepdims=True)
        out_ref[...] = ((yt - mu) * jax.lax.rsqrt(var + 1e-5)
                        * gamma_ref[...] + beta_ref[...])


def kernel(x, W_color, b_color, Wb1, bb1, bw1, Wb2, bb2, bw2, Wb3, bb3, bw3,
           We, be, gamma, beta):
    x2 = x.reshape(_S, _D)
    outt = pl.pallas_call(
        _body,
        grid=(_K,),
        in_specs=[
            pl.BlockSpec((_S, _D), lambda k: (0, 0)),
            pl.BlockSpec((_D, _SD), lambda k: (0, 0)),
            pl.BlockSpec((1, _SD), lambda k: (0, 0)),
            pl.BlockSpec((_NB, _BSZ, _ND), lambda k: (0, 0, 0)),
            pl.BlockSpec((_NB, _ND), lambda k: (0, 0)),
            pl.BlockSpec((1, _NB), lambda k: (0, 0)),
            pl.BlockSpec((_NB, _BSZ, _ND * _NS), lambda k: (0, 0, 0)),
            pl.BlockSpec((_NB, _ND * _NS), lambda k: (0, 0)),
            pl.BlockSpec((1, _NB), lambda k: (0, 0)),
            pl.BlockSpec((_NB, _BSZ, _K), lambda k: (0, 0, 0)),
            pl.BlockSpec((_NB, _K), lambda k: (0, 0)),
            pl.BlockSpec((1, _NB), lambda k: (0, 0)),
            pl.BlockSpec((1, _D, _D), lambda k: (k, 0, 0)),
            pl.BlockSpec((_D, _K), lambda k: (0, 0)),
            pl.BlockSpec((_D, 1), lambda k: (0, 0)),
            pl.BlockSpec((_D, 1), lambda k: (0, 0)),
            pl.BlockSpec((_D, _S), lambda k: (0, 0)),
        ],
        out_specs=pl.BlockSpec((_D, _S), lambda k: (0, 0)),
        out_shape=jax.ShapeDtypeStruct((_D, _S), jnp.float32),
        scratch_shapes=[
            pltpu.VMEM((_D, _S), jnp.float8_e4m3fn),
            pltpu.VMEM((_K, _S), jnp.bfloat16),
            pltpu.VMEM((_K, 8, _S), jnp.float32),
        ],
        compiler_params=pltpu.CompilerParams(
            dimension_semantics=("arbitrary",),
            vmem_limit_bytes=63 * 1024 * 1024,
        ),
    )(x2, W_color, b_color.reshape(1, _SD), Wb1, bb1, bw1.reshape(1, _NB),
      Wb2, bb2, bw2.reshape(1, _NB), Wb3, bb3, bw3.reshape(1, _NB),
      We, be.T, gamma.reshape(_D, 1), beta.reshape(_D, 1), x2.T)
    return outt.T.reshape(1, _S, _D)
